# baseline (speedup 1.0000x reference)
import functools

import jax
import jax.numpy as jnp
from jax.experimental import pallas as pl

_NPOINT = 1024
_RADII = (0.1, 0.2, 0.4)
_KS = (16, 32, 64)
_EPS = 1e-5
_SBLK = 128
_NC = 128
_CPAD = 8


def _dot(a, b, dims):
    return jax.lax.dot_general(a, b, (dims, ((), ())),
                               preferred_element_type=jnp.float32)


def _fps_kernel(xyz_ref, newxyz_ref, *, npoint):
    b, _, n = xyz_ref.shape
    sub, lanes = 8, n // 8
    iota_s = jax.lax.broadcasted_iota(jnp.int32, (b, npoint), 1)
    nidx = (jax.lax.broadcasted_iota(jnp.int32, (sub, lanes), 0) * lanes +
            jax.lax.broadcasted_iota(jnp.int32, (sub, lanes), 1))
    xs = [xyz_ref[bi, 0, :].reshape(sub, lanes) for bi in range(b)]
    ys = [xyz_ref[bi, 1, :].reshape(sub, lanes) for bi in range(b)]
    zs = [xyz_ref[bi, 2, :].reshape(sub, lanes) for bi in range(b)]

    def body(i, state):
        dists, fars, nxa, nya, nza = state
        cxl, cyl, czl = [], [], []
        new_dists, new_fars = [], []
        for bi in range(b):
            sel = nidx == fars[bi]
            cx = jnp.sum(jnp.where(sel, xs[bi], 0.0))
            cy = jnp.sum(jnp.where(sel, ys[bi], 0.0))
            cz = jnp.sum(jnp.where(sel, zs[bi], 0.0))
            d = ((xs[bi] - cx) ** 2 + (ys[bi] - cy) ** 2
                 + (zs[bi] - cz) ** 2)
            dist = jnp.minimum(dists[bi], d)
            m = jnp.max(dist)
            nf = jnp.min(jnp.where(dist == m, nidx, n))
            new_dists.append(dist)
            new_fars.append(nf)
            cxl.append(cx.reshape(1, 1))
            cyl.append(cy.reshape(1, 1))
            czl.append(cz.reshape(1, 1))
        wr = iota_s == i
        nxa = jnp.where(wr, jnp.concatenate(cxl, axis=0), nxa)
        nya = jnp.where(wr, jnp.concatenate(cyl, axis=0), nya)
        nza = jnp.where(wr, jnp.concatenate(czl, axis=0), nza)
        return tuple(new_dists), tuple(new_fars), nxa, nya, nza

    dist0 = tuple(jnp.full((sub, lanes), 1e10, jnp.float32)
                  for _ in range(b))
    far0 = tuple(jnp.zeros((), jnp.int32) for _ in range(b))
    acc0 = jnp.zeros((b, npoint), jnp.float32)
    _, _, nxa, nya, nza = jax.lax.fori_loop(
        0, npoint, body, (dist0, far0, acc0, acc0, acc0))
    newxyz_ref[:, 0, :] = nxa
    newxyz_ref[:, 1, :] = nya
    newxyz_ref[:, 2, :] = nza


def _cumsum_lanes(mbf, nc):
    sblk, n = mbf.shape
    tri = (jax.lax.broadcasted_iota(jnp.int32, (nc, nc), 0) <=
           jax.lax.broadcasted_iota(jnp.int32, (nc, nc), 1)
           ).astype(jnp.bfloat16)
    parts = []
    carry = jnp.zeros((sblk, 1), jnp.float32)
    for c in range(n // nc):
        cs = _dot(mbf[:, c * nc:(c + 1) * nc], tri, ((1,), (0,)))
        parts.append(cs + carry)
        carry = carry + cs[:, nc - 1:nc]
    return jnp.concatenate(parts, axis=1)


def _group_kernel(phi_ref, plo_ref, xyz_ref, nxt_ref, nx3_ref,
                  x1_ref, x2_ref, x3_ref, g1_ref, g2_ref, g3_ref):
    phi = phi_ref[...]
    plo = plo_ref[...]
    nxt = nxt_ref[...]
    nx3 = nx3_ref[...]
    n = phi.shape[1]
    xyz = xyz_ref[...]
    cn = jnp.sum(nxt * nxt, axis=1, keepdims=True)
    pn = jnp.sum(xyz * xyz, axis=0, keepdims=True)
    cross = _dot(nxt, xyz, ((1,), (0,)))
    d2 = -2.0 * cross
    d2 = d2 + cn
    d2 = d2 + pn

    ones_row = jnp.ones((1, n), jnp.float32)
    one_bf = jnp.ones((), jnp.bfloat16)
    zero_bf = jnp.zeros((), jnp.bfloat16)
    first = (pl.program_id(0) == 0) & (pl.program_id(1) == 0)

    @pl.when(first)
    def _():
        g1_ref[...] = jnp.zeros_like(g1_ref)
        g2_ref[...] = jnp.zeros_like(g2_ref)
        g3_ref[...] = jnp.zeros_like(g3_ref)

    for radius, nsample, x_ref, g_ref in (
            (_RADII[0], _KS[0], x1_ref, g1_ref),
            (_RADII[1], _KS[1], x2_ref, g2_ref),
            (_RADII[2], _KS[2], x3_ref, g3_ref)):
        maskf = (d2 <= radius * radius).astype(jnp.float32)
        t_row = _dot(ones_row, maskf, ((1,), (1,)))
        mbf = maskf.astype(jnp.bfloat16)
        rank = _cumsum_lanes(mbf, _NC)
        ranksel = (jnp.minimum(rank, float(nsample + 1)) * maskf
                   ).astype(jnp.bfloat16)
        h0 = jnp.where(ranksel == one_bf, one_bf, zero_bf)
        g0 = _dot(phi, h0, ((1,), (1,))) + _dot(plo, h0, ((1,), (1,)))

        def body(k, G, x_ref=x_ref, ranksel=ranksel, g0=g0, t_row=t_row):
            kp1f = (k + 1).astype(jnp.float32)
            kp1 = kp1f.astype(jnp.bfloat16)
            h = jnp.where(ranksel == kp1, one_bf, zero_bf)
            x0k = _dot(phi, h, ((1,), (1,))) + _dot(plo, h, ((1,), (1,)))
            padrow = (t_row < kp1f).astype(jnp.float32)
            x0k = x0k + g0 * padrow
            x0k = jnp.concatenate(
                [x0k[0:3], x0k[3:6] - nx3, x0k[6:8]], axis=0)
            x_ref[pl.ds(k, 1)] = x0k[None]
            return G + _dot(x0k, x0k, ((1,), (1,)))

        G = jax.lax.fori_loop(0, nsample, body,
                              jnp.zeros((_CPAD, _CPAD), jnp.float32))
        g_ref[...] += G


def _stats_pass_kernel(x_ref, *args, nlayers):
    g_ref = args[-1]
    ws = args[:-1]
    k_tot, _, s = x_ref.shape
    caug = g_ref.shape[0]
    extra = (jax.lax.broadcasted_iota(jnp.int32, (8, s), 0) == 0
             ).astype(jnp.float32)

    def body(k, G):
        y = x_ref[pl.ds(k, 1)][0]
        for j in range(nlayers):
            w = ws[2 * j][...]
            bb = ws[2 * j + 1][...]
            y = jnp.maximum(_dot(w, y, ((1,), (0,))) + bb, 0.0)
        ya = jnp.concatenate([y, extra], axis=0)
        return G + _dot(ya, ya, ((1,), (1,)))

    G = jax.lax.fori_loop(0, k_tot, body,
                          jnp.zeros((caug, caug), jnp.float32))

    @pl.when(pl.program_id(0) == 0)
    def _():
        g_ref[...] = jnp.zeros_like(g_ref)

    g_ref[...] += G


def _final_pass_kernel(x_ref, w1_ref, b1_ref, w2_ref, b2_ref, w3_ref, b3_ref,
                       out_ref):
    k_tot = x_ref.shape[0]
    out_ref[...] = jnp.full_like(out_ref, -1e30)

    def body(k, carry):
        y = x_ref[pl.ds(k, 1)][0]
        y = jnp.maximum(_dot(w1_ref[...], y, ((1,), (0,))) + b1_ref[...], 0.0)
        y = jnp.maximum(_dot(w2_ref[...], y, ((1,), (0,))) + b2_ref[...], 0.0)
        y = jnp.maximum(_dot(w3_ref[...], y, ((1,), (0,))) + b3_ref[...], 0.0)
        out_ref[...] = jnp.maximum(out_ref[...], y)
        return carry

    jax.lax.fori_loop(0, k_tot, body, 0)


def _fold(W, bias, gamma, beta, Gn, m):
    zm = W @ m + bias
    Ez2 = jnp.einsum('ci,ij,cj->c', W, Gn, W) + 2.0 * bias * (W @ m) \
        + bias * bias
    var = Ez2 - zm * zm
    a = gamma / jnp.sqrt(var + _EPS)
    return W * a[:, None], (a * (bias - zm) + beta)[:, None]


def kernel(xyz, points, params):
    b, _, n = xyz.shape
    s = _NPOINT
    cin = points.shape[1] + 3

    newxyz = pl.pallas_call(
        functools.partial(_fps_kernel, npoint=s),
        out_shape=jax.ShapeDtypeStruct((b, 3, s), jnp.float32),
    )(xyz)

    nxt = newxyz.transpose(0, 2, 1)
    p_tab = jnp.concatenate(
        [points, xyz, jnp.ones((b, 1, n), jnp.float32),
         jnp.zeros((b, 1, n), jnp.float32)], axis=1)
    p_hi = p_tab.astype(jnp.bfloat16)
    p_lo = (p_tab - p_hi.astype(jnp.float32)).astype(jnp.bfloat16)

    nblk = s // _SBLK
    x_shapes = [jax.ShapeDtypeStruct((b, k, _CPAD, s), jnp.float32)
                for k in _KS]
    g_shapes = [jax.ShapeDtypeStruct((_CPAD, _CPAD), jnp.float32)] * 3
    xs_and_gs = pl.pallas_call(
        _group_kernel,
        grid=(b, nblk),
        in_specs=[
            pl.BlockSpec((None, _CPAD, n), lambda bi, sb: (bi, 0, 0)),
            pl.BlockSpec((None, _CPAD, n), lambda bi, sb: (bi, 0, 0)),
            pl.BlockSpec((None, 3, n), lambda bi, sb: (bi, 0, 0)),
            pl.BlockSpec((None, _SBLK, 3), lambda bi, sb: (bi, sb, 0)),
            pl.BlockSpec((None, 3, _SBLK), lambda bi, sb: (bi, 0, sb)),
        ],
        out_specs=[
            pl.BlockSpec((None, _KS[0], _CPAD, _SBLK),
                         lambda bi, sb: (bi, 0, 0, sb)),
            pl.BlockSpec((None, _KS[1], _CPAD, _SBLK),
                         lambda bi, sb: (bi, 0, 0, sb)),
            pl.BlockSpec((None, _KS[2], _CPAD, _SBLK),
                         lambda bi, sb: (bi, 0, 0, sb)),
            pl.BlockSpec((_CPAD, _CPAD), lambda bi, sb: (0, 0)),
            pl.BlockSpec((_CPAD, _CPAD), lambda bi, sb: (0, 0)),
            pl.BlockSpec((_CPAD, _CPAD), lambda bi, sb: (0, 0)),
        ],
        out_shape=x_shapes + g_shapes,
    )(p_hi, p_lo, xyz, nxt, newxyz)
    xs, gs = xs_and_gs[:3], xs_and_gs[3:]

    outs = []
    for i, (ksamp, g0) in enumerate(zip(_KS, gs)):
        m_tot = float(b * ksamp * s)
        Gn = g0[0:cin, 0:cin] / m_tot
        mv = g0[0:cin, cin] / m_tot
        folded = []
        cout = cin
        for j in range(3):
            W = params['w_%d_%d' % (i, j)]
            bias = params['b_%d_%d' % (i, j)]
            gamma = params['g_%d_%d' % (i, j)]
            beta = params['be_%d_%d' % (i, j)]
            Wf, bf = _fold(W, bias, gamma, beta, Gn, mv)
            if j == 0:
                Wf = jnp.concatenate(
                    [Wf, jnp.zeros((Wf.shape[0], _CPAD - cin), jnp.float32)],
                    axis=1)
            folded += [Wf, bf]
            cout = W.shape[0]
            if j < 2:
                caug = cout + 8
                in_specs = [pl.BlockSpec((None, ksamp, _CPAD, s),
                                         lambda bi: (bi, 0, 0, 0))]
                for arr in folded:
                    in_specs.append(
                        pl.BlockSpec(arr.shape, lambda bi: (0, 0)))
                ga = pl.pallas_call(
                    functools.partial(_stats_pass_kernel, nlayers=j + 1),
                    grid=(b,),
                    in_specs=in_specs,
                    out_specs=pl.BlockSpec((caug, caug), lambda bi: (0, 0)),
                    out_shape=jax.ShapeDtypeStruct((caug, caug), jnp.float32),
                )(xs[i], *folded)
                Gn = ga[0:cout, 0:cout] / m_tot
                mv = ga[0:cout, cout] / m_tot
        in_specs = [pl.BlockSpec((None, ksamp, _CPAD, s),
                                 lambda bi: (bi, 0, 0, 0))]
        for arr in folded:
            in_specs.append(pl.BlockSpec(arr.shape, lambda bi: (0, 0)))
        out_i = pl.pallas_call(
            _final_pass_kernel,
            grid=(b,),
            in_specs=in_specs,
            out_specs=pl.BlockSpec((None, cout, s), lambda bi: (bi, 0, 0)),
            out_shape=jax.ShapeDtypeStruct((b, cout, s), jnp.float32),
        )(xs[i], *folded)
        outs.append(out_i)

    return newxyz, jnp.concatenate(outs, axis=1)


# speedup vs baseline: 1.6658x; 1.6658x over previous
import functools

import jax
import jax.numpy as jnp
from jax.experimental import pallas as pl

_NPOINT = 1024
_RADII = (0.1, 0.2, 0.4)
_KS = (16, 32, 64)
_EPS = 1e-5
_SBLK = 128
_NC = 128
_CPAD = 8


def _dot(a, b, dims):
    return jax.lax.dot_general(a, b, (dims, ((), ())),
                               preferred_element_type=jnp.float32)


def _fps_kernel(xyz_ref, newxyz_ref, *, npoint):
    b, _, n = xyz_ref.shape
    x = xyz_ref[:, 0, :]
    y = xyz_ref[:, 1, :]
    z = xyz_ref[:, 2, :]
    iota_n = jax.lax.broadcasted_iota(jnp.int32, (b, n), 1)
    iota_s = jax.lax.broadcasted_iota(jnp.int32, (b, npoint), 1)

    def body(i, state):
        dist, far, nxa, nya, nza = state
        sel = iota_n == far
        cx = jnp.sum(jnp.where(sel, x, 0.0), axis=1, keepdims=True)
        cy = jnp.sum(jnp.where(sel, y, 0.0), axis=1, keepdims=True)
        cz = jnp.sum(jnp.where(sel, z, 0.0), axis=1, keepdims=True)
        d = (x - cx) ** 2 + (y - cy) ** 2 + (z - cz) ** 2
        dist = jnp.minimum(dist, d)
        m = jnp.max(dist, axis=1, keepdims=True)
        far = jnp.min(jnp.where(dist == m, iota_n, n), axis=1,
                      keepdims=True).astype(jnp.int32)
        wr = iota_s == i
        nxa = jnp.where(wr, cx, nxa)
        nya = jnp.where(wr, cy, nya)
        nza = jnp.where(wr, cz, nza)
        return dist, far, nxa, nya, nza

    dist0 = jnp.full((b, n), 1e10, jnp.float32)
    far0 = jnp.zeros((b, 1), jnp.int32)
    acc0 = jnp.zeros((b, npoint), jnp.float32)
    _, _, nxa, nya, nza = jax.lax.fori_loop(
        0, npoint, body, (dist0, far0, acc0, acc0, acc0))
    newxyz_ref[:, 0, :] = nxa
    newxyz_ref[:, 1, :] = nya
    newxyz_ref[:, 2, :] = nza


def _cumsum_lanes(mbf, nc):
    sblk, n = mbf.shape
    tri = (jax.lax.broadcasted_iota(jnp.int32, (nc, nc), 0) <=
           jax.lax.broadcasted_iota(jnp.int32, (nc, nc), 1)
           ).astype(jnp.bfloat16)
    parts = []
    carry = jnp.zeros((sblk, 1), jnp.float32)
    for c in range(n // nc):
        cs = _dot(mbf[:, c * nc:(c + 1) * nc], tri, ((1,), (0,)))
        parts.append(cs + carry)
        carry = carry + cs[:, nc - 1:nc]
    return jnp.concatenate(parts, axis=1)


def _group_kernel(p_ref, nxt_ref, nx3_ref,
                  x1_ref, x2_ref, x3_ref, g1_ref, g2_ref, g3_ref):
    p = p_ref[...]
    nxt = nxt_ref[...]
    nx3 = nx3_ref[...]
    n = p.shape[1]
    xyz = p[3:6, :]
    cn = jnp.sum(nxt * nxt, axis=1, keepdims=True)
    pn = jnp.sum(xyz * xyz, axis=0, keepdims=True)
    cross = _dot(nxt, xyz, ((1,), (0,)))
    d2 = -2.0 * cross
    d2 = d2 + cn
    d2 = d2 + pn

    ones_row = jnp.ones((1, n), jnp.float32)
    first = (pl.program_id(0) == 0) & (pl.program_id(1) == 0)

    @pl.when(first)
    def _():
        g1_ref[...] = jnp.zeros_like(g1_ref)
        g2_ref[...] = jnp.zeros_like(g2_ref)
        g3_ref[...] = jnp.zeros_like(g3_ref)

    for radius, nsample, x_ref, g_ref in (
            (_RADII[0], _KS[0], x1_ref, g1_ref),
            (_RADII[1], _KS[1], x2_ref, g2_ref),
            (_RADII[2], _KS[2], x3_ref, g3_ref)):
        maskf = (d2 <= radius * radius).astype(jnp.float32)
        t_row = _dot(ones_row, maskf, ((1,), (1,)))
        mbf = maskf.astype(jnp.bfloat16)
        rank = _cumsum_lanes(mbf, _NC)
        ranksel = rank * maskf
        h0 = jnp.where(ranksel == 1.0, 1.0, 0.0)
        g0 = _dot(p, h0, ((1,), (1,)))

        def body(k, G, x_ref=x_ref, ranksel=ranksel, g0=g0, t_row=t_row):
            kp1f = (k + 1).astype(jnp.float32)
            h = jnp.where(ranksel == kp1f, 1.0, 0.0)
            x0k = _dot(p, h, ((1,), (1,)))
            padrow = (t_row < kp1f).astype(jnp.float32)
            x0k = x0k + g0 * padrow
            x0k = jnp.concatenate(
                [x0k[0:3], x0k[3:6] - nx3, x0k[6:8]], axis=0)
            x_ref[pl.ds(k, 1)] = x0k[None]
            return G + _dot(x0k, x0k, ((1,), (1,)))

        G = jax.lax.fori_loop(0, nsample, body,
                              jnp.zeros((_CPAD, _CPAD), jnp.float32))
        g_ref[...] += G


def _stats_pass_kernel(x_ref, *args, nlayers):
    g_ref = args[-1]
    ws = args[:-1]
    k_tot, _, s = x_ref.shape
    caug = g_ref.shape[0]
    extra = (jax.lax.broadcasted_iota(jnp.int32, (8, s), 0) == 0
             ).astype(jnp.float32)

    def body(k, G):
        y = x_ref[pl.ds(k, 1)][0]
        for j in range(nlayers):
            w = ws[2 * j][...]
            bb = ws[2 * j + 1][...]
            y = jnp.maximum(_dot(w, y, ((1,), (0,))) + bb, 0.0)
        ya = jnp.concatenate([y, extra], axis=0)
        return G + _dot(ya, ya, ((1,), (1,)))

    G = jax.lax.fori_loop(0, k_tot, body,
                          jnp.zeros((caug, caug), jnp.float32))

    @pl.when(pl.program_id(0) == 0)
    def _():
        g_ref[...] = jnp.zeros_like(g_ref)

    g_ref[...] += G


def _final_pass_kernel(x_ref, w1_ref, b1_ref, w2_ref, b2_ref, w3_ref, b3_ref,
                       out_ref):
    k_tot = x_ref.shape[0]
    out_ref[...] = jnp.full_like(out_ref, -1e30)

    def body(k, carry):
        y = x_ref[pl.ds(k, 1)][0]
        y = jnp.maximum(_dot(w1_ref[...], y, ((1,), (0,))) + b1_ref[...], 0.0)
        y = jnp.maximum(_dot(w2_ref[...], y, ((1,), (0,))) + b2_ref[...], 0.0)
        y = jnp.maximum(_dot(w3_ref[...], y, ((1,), (0,))) + b3_ref[...], 0.0)
        out_ref[...] = jnp.maximum(out_ref[...], y)
        return carry

    jax.lax.fori_loop(0, k_tot, body, 0)


def _fold(W, bias, gamma, beta, Gn, m):
    zm = W @ m + bias
    Ez2 = jnp.einsum('ci,ij,cj->c', W, Gn, W) + 2.0 * bias * (W @ m) \
        + bias * bias
    var = Ez2 - zm * zm
    a = gamma / jnp.sqrt(var + _EPS)
    return W * a[:, None], (a * (bias - zm) + beta)[:, None]


def kernel(xyz, points, params):
    b, _, n = xyz.shape
    s = _NPOINT
    cin = points.shape[1] + 3

    newxyz = pl.pallas_call(
        functools.partial(_fps_kernel, npoint=s),
        out_shape=jax.ShapeDtypeStruct((b, 3, s), jnp.float32),
    )(xyz)

    nxt = newxyz.transpose(0, 2, 1)
    p_tab = jnp.concatenate(
        [points, xyz, jnp.ones((b, 1, n), jnp.float32),
         jnp.zeros((b, 1, n), jnp.float32)], axis=1)

    nblk = s // _SBLK
    x_shapes = [jax.ShapeDtypeStruct((b, k, _CPAD, s), jnp.float32)
                for k in _KS]
    g_shapes = [jax.ShapeDtypeStruct((_CPAD, _CPAD), jnp.float32)] * 3
    xs_and_gs = pl.pallas_call(
        _group_kernel,
        grid=(b, nblk),
        in_specs=[
            pl.BlockSpec((None, _CPAD, n), lambda bi, sb: (bi, 0, 0)),
            pl.BlockSpec((None, _SBLK, 3), lambda bi, sb: (bi, sb, 0)),
            pl.BlockSpec((None, 3, _SBLK), lambda bi, sb: (bi, 0, sb)),
        ],
        out_specs=[
            pl.BlockSpec((None, _KS[0], _CPAD, _SBLK),
                         lambda bi, sb: (bi, 0, 0, sb)),
            pl.BlockSpec((None, _KS[1], _CPAD, _SBLK),
                         lambda bi, sb: (bi, 0, 0, sb)),
            pl.BlockSpec((None, _KS[2], _CPAD, _SBLK),
                         lambda bi, sb: (bi, 0, 0, sb)),
            pl.BlockSpec((_CPAD, _CPAD), lambda bi, sb: (0, 0)),
            pl.BlockSpec((_CPAD, _CPAD), lambda bi, sb: (0, 0)),
            pl.BlockSpec((_CPAD, _CPAD), lambda bi, sb: (0, 0)),
        ],
        out_shape=x_shapes + g_shapes,
    )(p_tab, nxt, newxyz)
    xs, gs = xs_and_gs[:3], xs_and_gs[3:]

    outs = []
    for i, (ksamp, g0) in enumerate(zip(_KS, gs)):
        m_tot = float(b * ksamp * s)
        Gn = g0[0:cin, 0:cin] / m_tot
        mv = g0[0:cin, cin] / m_tot
        folded = []
        cout = cin
        for j in range(3):
            W = params['w_%d_%d' % (i, j)]
            bias = params['b_%d_%d' % (i, j)]
            gamma = params['g_%d_%d' % (i, j)]
            beta = params['be_%d_%d' % (i, j)]
            Wf, bf = _fold(W, bias, gamma, beta, Gn, mv)
            if j == 0:
                Wf = jnp.concatenate(
                    [Wf, jnp.zeros((Wf.shape[0], _CPAD - cin), jnp.float32)],
                    axis=1)
            folded += [Wf, bf]
            cout = W.shape[0]
            if j < 2:
                caug = cout + 8
                in_specs = [pl.BlockSpec((None, ksamp, _CPAD, s),
                                         lambda bi: (bi, 0, 0, 0))]
                for arr in folded:
                    in_specs.append(
                        pl.BlockSpec(arr.shape, lambda bi: (0, 0)))
                ga = pl.pallas_call(
                    functools.partial(_stats_pass_kernel, nlayers=j + 1),
                    grid=(b,),
                    in_specs=in_specs,
                    out_specs=pl.BlockSpec((caug, caug), lambda bi: (0, 0)),
                    out_shape=jax.ShapeDtypeStruct((caug, caug), jnp.float32),
                )(xs[i], *folded)
                Gn = ga[0:cout, 0:cout] / m_tot
                mv = ga[0:cout, cout] / m_tot
        in_specs = [pl.BlockSpec((None, ksamp, _CPAD, s),
                                 lambda bi: (bi, 0, 0, 0))]
        for arr in folded:
            in_specs.append(pl.BlockSpec(arr.shape, lambda bi: (0, 0)))
        out_i = pl.pallas_call(
            _final_pass_kernel,
            grid=(b,),
            in_specs=in_specs,
            out_specs=pl.BlockSpec((None, cout, s), lambda bi: (bi, 0, 0)),
            out_shape=jax.ShapeDtypeStruct((b, cout, s), jnp.float32),
        )(xs[i], *folded)
        outs.append(out_i)

    return newxyz, jnp.concatenate(outs, axis=1)
